# Initial kernel scaffold; baseline (speedup 1.0000x reference)
import jax
import jax.numpy as jnp
from jax.experimental import pallas as pl


def kernel(raw_wav):
    raise NotImplementedError("write your pallas kernel here")


# speedup vs baseline: 2.9260x; 2.9260x over previous
import functools

import jax
import jax.numpy as jnp
from jax import lax
from jax.experimental import pallas as pl
from jax.experimental.pallas import tpu as pltpu
from jax.experimental.pallas import tpu_sc as plsc

_SHIFT = 8192
_B, _CH, _LEN = 64, 2, 160000
_OUT = _LEN - _SHIFT
_ROWS = _B * _CH
_NW = 32
_RPW = _ROWS // _NW
_NCHUNK = 4
_CHUNK = _OUT // _NCHUNK


@functools.partial(
    pl.kernel,
    out_type=jax.ShapeDtypeStruct((_ROWS * _OUT,), jnp.float32),
    mesh=plsc.VectorSubcoreMesh(core_axis_name="c", subcore_axis_name="s"),
    scratch_types=[
        pltpu.VMEM((16,), jnp.int32),
        pltpu.VMEM((_CHUNK + 16,), jnp.float32),
        pltpu.VMEM((_CHUNK,), jnp.float32),
    ],
)
def _shift_copy(wav_hbm, base_hbm, out_hbm, row_v, buf, buf2):
    wid = lax.axis_index("s") * 2 + lax.axis_index("c")
    pltpu.sync_copy(base_hbm.at[wid], row_v)
    bases = row_v[...]
    for k in range(_RPW):
        src = bases[k]
        rlo = lax.bitwise_and(src, 7)
        src_al = pl.multiple_of(src - rlo, 8)
        dst = (wid * _RPW + k) * _OUT
        for j in range(_NCHUNK):
            pltpu.sync_copy(
                wav_hbm.at[pl.ds(src_al + j * _CHUNK, _CHUNK + 16)], buf)

            def realign(i, _):
                buf2[pl.ds(i * 16, 16)] = buf[pl.ds(i * 16 + rlo, 16)]
                return 0

            lax.fori_loop(0, _CHUNK // 16, realign, 0)
            pltpu.sync_copy(
                buf2, out_hbm.at[pl.ds(dst + j * _CHUNK, _CHUNK)])


def kernel(raw_wav):
    okey = jax.random.key(42)
    offsets = jax.random.randint(okey, (_B, 1, 1), 0, _SHIFT)
    offs = offsets.reshape(_B).astype(jnp.int32)
    row_off = jnp.repeat(offs, _CH)
    src_base = jnp.arange(_ROWS, dtype=jnp.int32) * _LEN + row_off
    base_tbl = jnp.zeros((_NW, 16), jnp.int32)
    base_tbl = base_tbl.at[:, :_RPW].set(src_base.reshape(_NW, _RPW))
    out_flat = _shift_copy(raw_wav.reshape(-1), base_tbl)
    return out_flat.reshape(_B, _CH, _OUT)


# speedup vs baseline: 4.3903x; 1.5005x over previous
import functools

import jax
import jax.numpy as jnp
from jax import lax
from jax.experimental import pallas as pl
from jax.experimental.pallas import tpu as pltpu
from jax.experimental.pallas import tpu_sc as plsc

_SHIFT = 8192
_B, _CH, _LEN = 64, 2, 160000
_OUT = _LEN - _SHIFT
_ROWS = _B * _CH
_NW = 32
_RPW = _ROWS // _NW
_NCHUNK = 8
_CHUNK = _OUT // _NCHUNK
_IN_N = _CHUNK + 8
_UNROLL = 8
_CVEC = _CHUNK // 16
_CVEC_PAD = -(-_CVEC // _UNROLL) * _UNROLL
_INBUF = _CVEC_PAD * 16 + 16
_OUTBUF = _CVEC_PAD * 16
_T = _RPW * _NCHUNK


@functools.partial(
    pl.kernel,
    out_type=jax.ShapeDtypeStruct((_ROWS * _OUT,), jnp.float32),
    mesh=plsc.VectorSubcoreMesh(core_axis_name="c", subcore_axis_name="s"),
    scratch_types=[
        pltpu.VMEM((16,), jnp.int32),
        pltpu.VMEM((_INBUF,), jnp.float32),
        pltpu.VMEM((_INBUF,), jnp.float32),
        pltpu.VMEM((_OUTBUF,), jnp.float32),
        pltpu.VMEM((_OUTBUF,), jnp.float32),
        pltpu.SemaphoreType.DMA,
        pltpu.SemaphoreType.DMA,
        pltpu.SemaphoreType.DMA,
        pltpu.SemaphoreType.DMA,
    ],
)
def _shift_copy(wav_hbm, base_hbm, out_hbm, row_v,
                in0, in1, out0, out1, si0, si1, so0, so1):
    wid = lax.axis_index("s") * 2 + lax.axis_index("c")
    pltpu.sync_copy(base_hbm.at[wid], row_v)
    bases = row_v[...]
    ins, outs = (in0, in1), (out0, out1)
    sin, sout = (si0, si1), (so0, so1)

    def params(t):
        k, j = divmod(t, _NCHUNK)
        src = bases[k]
        rlo = lax.bitwise_and(src, 7)
        src_al = pl.multiple_of(src - rlo, 8)
        dst = (wid * _RPW + k) * _OUT + j * _CHUNK
        return src_al + j * _CHUNK, rlo, dst

    def start_in(t):
        s, _, _ = params(t)
        return pltpu.async_copy(
            wav_hbm.at[pl.ds(s, _IN_N)],
            ins[t % 2].at[pl.ds(0, _IN_N)], sin[t % 2])

    in_h, out_h = {}, {}
    in_h[0] = start_in(0)
    in_h[1] = start_in(1)
    for t in range(_T):
        _, rlo, dst = params(t)
        in_h[t].wait()
        if t >= 2:
            out_h[t - 2].wait()
        ib, ob = ins[t % 2], outs[t % 2]

        @plsc.parallel_loop(0, _CVEC_PAD, step=_UNROLL)
        def _(i):
            for u in range(_UNROLL):
                ob[pl.ds((i + u) * 16, 16)] = ib[pl.ds((i + u) * 16 + rlo, 16)]

        out_h[t] = pltpu.async_copy(
            ob.at[pl.ds(0, _CHUNK)],
            out_hbm.at[pl.ds(dst, _CHUNK)], sout[t % 2])
        if t + 2 < _T:
            in_h[t + 2] = start_in(t + 2)
    out_h[_T - 2].wait()
    out_h[_T - 1].wait()


def kernel(raw_wav):
    okey = jax.random.key(42)
    offsets = jax.random.randint(okey, (_B, 1, 1), 0, _SHIFT)
    offs = offsets.reshape(_B).astype(jnp.int32)
    row_off = jnp.repeat(offs, _CH)
    src_base = jnp.arange(_ROWS, dtype=jnp.int32) * _LEN + row_off
    base_tbl = jnp.zeros((_NW, 16), jnp.int32)
    base_tbl = base_tbl.at[:, :_RPW].set(src_base.reshape(_NW, _RPW))
    out_flat = _shift_copy(raw_wav.reshape(-1), base_tbl)
    return out_flat.reshape(_B, _CH, _OUT)


# speedup vs baseline: 21.5518x; 4.9089x over previous
import functools

import jax
import jax.numpy as jnp
from jax import lax
from jax.experimental import pallas as pl
from jax.experimental.pallas import tpu as pltpu
from jax.experimental.pallas import tpu_sc as plsc

_SHIFT = 8192
_B, _CH, _LEN = 64, 2, 160000
_OUT = _LEN - _SHIFT
_ROWS = _B * _CH
_NW = 32
_RPW = _ROWS // _NW
_NT = _OUT // 128
_CHUNKS = [198, 198, 198, 198, 198, 196]
_MMAX = max(_CHUNKS)
_UNROLL = 8


@functools.partial(
    pl.kernel,
    out_type=jax.ShapeDtypeStruct((_B, _CH, _OUT), jnp.float32),
    mesh=plsc.VectorSubcoreMesh(core_axis_name="c", subcore_axis_name="s"),
    scratch_types=[
        pltpu.VMEM((16,), jnp.int32),
        pltpu.VMEM(((_MMAX + 1) * 128,), jnp.float32),
        pltpu.VMEM(((_MMAX + 1) * 128,), jnp.float32),
        pltpu.VMEM((_MMAX * 128,), jnp.float32),
        pltpu.VMEM((_MMAX * 128,), jnp.float32),
        pltpu.SemaphoreType.DMA,
        pltpu.SemaphoreType.DMA,
        pltpu.SemaphoreType.DMA,
        pltpu.SemaphoreType.DMA,
    ],
    compiler_params=pltpu.CompilerParams(use_tc_tiling_on_sc=True),
)
def _shift_copy(wav_hbm, base_hbm, out_hbm, row_v,
                in0, in1, out0, out1, si0, si1, so0, so1):
    wid = lax.axis_index("s") * 2 + lax.axis_index("c")
    pltpu.sync_copy(base_hbm.at[wid], row_v)
    offs = row_v[...]
    ins, outs = (in0, in1), (out0, out1)
    sin, sout = (si0, si1), (so0, so1)

    tasks = []
    for k in range(_RPW):
        lt0 = 0
        for m in _CHUNKS:
            tasks.append((k, lt0, m))
            lt0 += m
    _T = len(tasks)

    def params(t):
        k, lt0, m = tasks[t]
        off = offs[k]
        r128 = lax.bitwise_and(off, 127)
        t_al = pl.multiple_of(off - r128, 128)
        b = wid * (_RPW // _CH) + k // _CH
        c = k % _CH
        return b, c, t_al + lt0 * 128, r128, lt0 * 128, m

    def start_in(t):
        b, c, t_in0, _, _, m = params(t)
        return pltpu.async_copy(
            wav_hbm.at[b, c, pl.ds(t_in0, (m + 1) * 128)],
            ins[t % 2].at[pl.ds(0, (m + 1) * 128)], sin[t % 2])

    in_h, out_h = {}, {}
    in_h[0] = start_in(0)
    in_h[1] = start_in(1)
    for t in range(_T):
        b, c, _, r128, dst0, m = params(t)
        in_h[t].wait()
        if t >= 2:
            out_h[t - 2].wait()
        ib, ob = ins[t % 2], outs[t % 2]

        @plsc.parallel_loop(0, m * 8, step=_UNROLL)
        def _(i):
            for u in range(_UNROLL):
                ob[pl.ds((i + u) * 16, 16)] = ib[pl.ds((i + u) * 16 + r128, 16)]

        out_h[t] = pltpu.async_copy(
            ob.at[pl.ds(0, m * 128)],
            out_hbm.at[b, c, pl.ds(dst0, m * 128)], sout[t % 2])
        if t + 2 < _T:
            in_h[t + 2] = start_in(t + 2)
    out_h[_T - 2].wait()
    out_h[_T - 1].wait()


def kernel(raw_wav):
    okey = jax.random.key(42)
    offsets = jax.random.randint(okey, (_B, 1, 1), 0, _SHIFT)
    offs = offsets.reshape(_B).astype(jnp.int32)
    row_off = jnp.repeat(offs, _CH)
    base_tbl = jnp.zeros((_NW, 16), jnp.int32)
    base_tbl = base_tbl.at[:, :_RPW].set(row_off.reshape(_NW, _RPW))
    return _shift_copy(raw_wav, base_tbl)
